# baseline (speedup 1.0000x reference)
import functools

import jax
import jax.numpy as jnp
from jax import lax
from jax.experimental import pallas as pl
from jax.experimental.pallas import tpu as pltpu
from jax.experimental.pallas import tpu_sc as plsc

_NUM_ILLU = 100000
_B = 1048576
_NC = 2
_NS = 16
_NW = _NC * _NS
_B_PER_W = _B // _NW
_CHUNK = 4096
_N_SETS = 2
_N_CHUNKS = _B_PER_W // _CHUNK
_L = 16
_GROUPS = _CHUNK // _L
_T_SPLIT = 41600


def _body(e_hbm, idx_hbm, w_hbm, out_hbm,
          table_v, table_sh, idx_a, idx_b, e_a, e_b, o_a, o_b,
          tsem, t2sem, ia_sem, ib_sem, ea_sem, eb_sem, oa_sem, ob_sem):
    sid = lax.axis_index("s")
    wid = sid * _NC + lax.axis_index("c")
    base = wid * _B_PER_W

    @pl.when(sid == 0)
    def _load_shared():
        pltpu.sync_copy(w_hbm, table_sh)

    thbm = pltpu.async_copy(
        w_hbm.at[pl.ds(0, _T_SPLIT)], table_v.at[pl.ds(0, _T_SPLIT)], t2sem)

    idx_bufs = (idx_a, idx_b)
    e_bufs = (e_a, e_b)
    o_bufs = (o_a, o_b)
    isems = (ia_sem, ib_sem)
    esems = (ea_sem, eb_sem)
    osems = (oa_sem, ob_sem)

    def start_in(ci):
        s = ci % _N_SETS
        off = base + ci * _CHUNK
        ic = pltpu.async_copy(idx_hbm.at[pl.ds(off, _CHUNK)], idx_bufs[s], isems[s])
        ec = pltpu.async_copy(e_hbm.at[pl.ds(off, _CHUNK)], e_bufs[s], esems[s])
        return ic, ec

    pend_in = {0: start_in(0), 1: start_in(1)}
    pend_out = {}

    plsc.subcore_barrier()
    tcopy = pltpu.async_copy(
        table_sh.at[pl.ds(_T_SPLIT, _NUM_ILLU - _T_SPLIT)],
        table_v.at[pl.ds(_T_SPLIT, _NUM_ILLU - _T_SPLIT)], tsem)
    tcopy.wait()
    thbm.wait()

    for ci in range(_N_CHUNKS):
        s = ci % _N_SETS
        ic, ec = pend_in.pop(ci)
        ic.wait()
        ec.wait()
        if ci - _N_SETS in pend_out:
            pend_out.pop(ci - _N_SETS).wait()
        iv, ev, ov = idx_bufs[s], e_bufs[s], o_bufs[s]

        @plsc.parallel_loop(0, _GROUPS, 1, unroll=8)
        def _gather(gi, iv=iv, ev=ev, ov=ov):
            sl = pl.ds(gi * _L, _L)
            ov[sl] = plsc.load_gather(table_v, [iv[sl]]) * ev[sl]

        off = base + ci * _CHUNK
        pend_out[ci] = pltpu.async_copy(ov, out_hbm.at[pl.ds(off, _CHUNK)], osems[s])
        if ci + _N_SETS < _N_CHUNKS:
            pend_in[ci + _N_SETS] = start_in(ci + _N_SETS)

    for oc in pend_out.values():
        oc.wait()


@jax.jit
def kernel(E_in, idx, weights):
    mesh = plsc.VectorSubcoreMesh(core_axis_name="c", subcore_axis_name="s")
    run = functools.partial(
        pl.kernel,
        out_type=jax.ShapeDtypeStruct((_B,), jnp.float32),
        mesh=mesh,
        compiler_params=pltpu.CompilerParams(
            needs_layout_passes=False, skip_device_barrier=True),
        scratch_types=[
            pltpu.VMEM((_NUM_ILLU,), jnp.float32),
            pltpu.VMEM_SHARED((_NUM_ILLU,), jnp.float32),
            pltpu.VMEM((_CHUNK,), jnp.int32),
            pltpu.VMEM((_CHUNK,), jnp.int32),
            pltpu.VMEM((_CHUNK,), jnp.float32),
            pltpu.VMEM((_CHUNK,), jnp.float32),
            pltpu.VMEM((_CHUNK,), jnp.float32),
            pltpu.VMEM((_CHUNK,), jnp.float32),
            pltpu.SemaphoreType.DMA,
            pltpu.SemaphoreType.DMA,
            pltpu.SemaphoreType.DMA,
            pltpu.SemaphoreType.DMA,
            pltpu.SemaphoreType.DMA,
            pltpu.SemaphoreType.DMA,
            pltpu.SemaphoreType.DMA,
            pltpu.SemaphoreType.DMA,
        ],
    )(_body)
    return run(E_in, idx.astype(jnp.int32), weights)


# speedup vs baseline: 1.0975x; 1.0975x over previous
import functools

import jax
import jax.numpy as jnp
from jax import lax
from jax.experimental import pallas as pl
from jax.experimental.pallas import tpu as pltpu
from jax.experimental.pallas import tpu_sc as plsc

_NUM_ILLU = 100000
_B = 1048576
_NC = 2
_NS = 16
_NW = _NC * _NS
_B_PER_W = _B // _NW
_CHUNK = 4096
_N_SETS = 2
_N_CHUNKS = _B_PER_W // _CHUNK
_L = 16
_GROUPS = _CHUNK // _L


def _body(e_hbm, idx_hbm, w_hbm, out_hbm,
          table_v, table_sh, idx_a, idx_b, e_a, e_b, o_a, o_b,
          tsem, ia_sem, ib_sem, ea_sem, eb_sem, oa_sem, ob_sem):
    sid = lax.axis_index("s")
    wid = sid * _NC + lax.axis_index("c")
    base = wid * _B_PER_W

    idx_bufs = (idx_a, idx_b)
    e_bufs = (e_a, e_b)
    o_bufs = (o_a, o_b)
    isems = (ia_sem, ib_sem)
    esems = (ea_sem, eb_sem)
    osems = (oa_sem, ob_sem)

    def start_in(ci):
        s = ci % _N_SETS
        off = base + ci * _CHUNK
        ic = pltpu.async_copy(idx_hbm.at[pl.ds(off, _CHUNK)], idx_bufs[s], isems[s])
        ec = pltpu.async_copy(e_hbm.at[pl.ds(off, _CHUNK)], e_bufs[s], esems[s])
        return ic, ec

    start_in(0)
    start_in(1)

    @pl.when(sid == 0)
    def _load_shared():
        pltpu.sync_copy(w_hbm, table_sh)

    plsc.subcore_barrier()
    tcopy = pltpu.async_copy(table_sh, table_v, tsem)
    tcopy.wait()

    def pair(pi, c):
        for s in range(_N_SETS):
            ci = pi * _N_SETS + s
            off = base + ci * _CHUNK
            iv, ev, ov = idx_bufs[s], e_bufs[s], o_bufs[s]
            pltpu.make_async_copy(
                idx_hbm.at[pl.ds(off, _CHUNK)], iv, isems[s]).wait()
            pltpu.make_async_copy(
                e_hbm.at[pl.ds(off, _CHUNK)], ev, esems[s]).wait()

            @pl.when(pi > 0)
            def _wait_prev_out():
                pltpu.make_async_copy(
                    ov, out_hbm.at[pl.ds(off, _CHUNK)], osems[s]).wait()

            @plsc.parallel_loop(0, _GROUPS, 1, unroll=8)
            def _gather(gi, iv=iv, ev=ev, ov=ov):
                sl = pl.ds(gi * _L, _L)
                ov[sl] = plsc.load_gather(table_v, [iv[sl]]) * ev[sl]

            pltpu.async_copy(ov, out_hbm.at[pl.ds(off, _CHUNK)], osems[s])

            @pl.when(pi < _N_CHUNKS // _N_SETS - 1)
            def _start_next_in():
                off2 = off + _N_SETS * _CHUNK
                pltpu.async_copy(idx_hbm.at[pl.ds(off2, _CHUNK)], iv, isems[s])
                pltpu.async_copy(e_hbm.at[pl.ds(off2, _CHUNK)], ev, esems[s])
        return c

    lax.fori_loop(0, _N_CHUNKS // _N_SETS, pair, 0)
    for s in range(_N_SETS):
        off = base + (_N_CHUNKS - _N_SETS + s) * _CHUNK
        pltpu.make_async_copy(
            o_bufs[s], out_hbm.at[pl.ds(off, _CHUNK)], osems[s]).wait()


@jax.jit
def kernel(E_in, idx, weights):
    mesh = plsc.VectorSubcoreMesh(core_axis_name="c", subcore_axis_name="s")
    run = functools.partial(
        pl.kernel,
        out_type=jax.ShapeDtypeStruct((_B,), jnp.float32),
        mesh=mesh,
        compiler_params=pltpu.CompilerParams(needs_layout_passes=False),
        scratch_types=[
            pltpu.VMEM((_NUM_ILLU,), jnp.float32),
            pltpu.VMEM_SHARED((_NUM_ILLU,), jnp.float32),
            pltpu.VMEM((_CHUNK,), jnp.int32),
            pltpu.VMEM((_CHUNK,), jnp.int32),
            pltpu.VMEM((_CHUNK,), jnp.float32),
            pltpu.VMEM((_CHUNK,), jnp.float32),
            pltpu.VMEM((_CHUNK,), jnp.float32),
            pltpu.VMEM((_CHUNK,), jnp.float32),
            pltpu.SemaphoreType.DMA,
            pltpu.SemaphoreType.DMA,
            pltpu.SemaphoreType.DMA,
            pltpu.SemaphoreType.DMA,
            pltpu.SemaphoreType.DMA,
            pltpu.SemaphoreType.DMA,
            pltpu.SemaphoreType.DMA,
        ],
    )(_body)
    return run(E_in, idx.astype(jnp.int32), weights)
